# baseline (speedup 1.0000x reference)
import jax
import jax.numpy as jnp
from jax.experimental import pallas as pl
from jax.experimental.pallas import tpu as pltpu

N = 10000
D = 128
H = 2048
OUT = 512
B = 1024


def _gcn_conv(x, W, b, src, dst, n_nodes):
    xw = x @ W
    deg = jnp.zeros((n_nodes,), dtype=xw.dtype).at[dst].add(1.0) + 1.0
    dinv = jax.lax.rsqrt(deg)
    coef = dinv[src] * dinv[dst]
    msg = xw[src] * coef[:, None]
    agg = jnp.zeros_like(xw).at[dst].add(msg)
    out = agg + xw * (1.0 / deg)[:, None] + b
    return out


def _l2_normalize(x):
    nrm = jnp.sqrt(jnp.sum(x * x, axis=1, keepdims=True))
    return x / jnp.maximum(nrm, 1e-12)


def _final_matmul_kernel(img_ref, cur_ref, out_ref):
    img = img_ref[...]
    cur = cur_ref[...]
    out_ref[...] = jax.lax.dot_general(
        img, cur, (((1,), (1,)), ((), ())),
        preferred_element_type=jnp.float32)


def _final_matmul(img_n, cur):
    NP = 10240
    TC = 1280
    cur_p = jnp.pad(cur, ((0, NP - N), (0, 0)))
    return pl.pallas_call(
        _final_matmul_kernel,
        grid=(NP // TC,),
        in_specs=[
            pl.BlockSpec((B, OUT), lambda j: (0, 0)),
            pl.BlockSpec((TC, OUT), lambda j: (j, 0)),
        ],
        out_specs=pl.BlockSpec((B, TC), lambda j: (0, j)),
        out_shape=jax.ShapeDtypeStruct((B, NP), jnp.float32),
    )(img_n, cur_p)


def kernel(img, nodes, edge_index, W1, b1, W2, b2, W3, b3):
    src = edge_index[0]
    dst = edge_index[1]
    h = _gcn_conv(nodes, W1, b1, src, dst, N)
    h = jax.nn.relu(h)
    h = _gcn_conv(h, W2, b2, src, dst, N)
    h = jax.nn.relu(h)
    h = _gcn_conv(h, W3, b3, src, dst, N)
    cur = _l2_normalize(h)
    img_n = _l2_normalize(img)
    pair_pred = _final_matmul(img_n, cur)[:, 500:N]
    return pair_pred


# speedup vs baseline: 1.5270x; 1.5270x over previous
import functools

import jax
import jax.numpy as jnp
from jax import lax
from jax.experimental import pallas as pl
from jax.experimental.pallas import tpu as pltpu
from jax.experimental.pallas import tpu_sc as plsc

N = 10000
D = 128
H = 2048
OUT = 512
B = 1024
E = 320000

C = 128
NW = 32
BT = 128
NB = 80
EPW = NB * BT
EPAD = NW * EPW
NROWS = 10240
RPT = NROWS // 16
NP = 10240


def _make_agg(K):
    mesh = plsc.VectorSubcoreMesh(core_axis_name="c", subcore_axis_name="s")

    @functools.partial(
        pl.kernel,
        out_type=jax.ShapeDtypeStruct((2 * K * NROWS, C), jnp.float32),
        mesh=mesh,
        scratch_types=[
            pltpu.VMEM((EPW,), jnp.int32),
            pltpu.VMEM((NB, BT), jnp.int32),
            pltpu.VMEM((BT, C), jnp.float32),
            pltpu.VMEM_SHARED((NROWS, C), jnp.float32),
            pltpu.SemaphoreType.DMA,
        ],
    )
    def agg(table, srcp, dstp, zeros_hbm, out, src_v, dst_v, buf, acc,
            sem):
        c = lax.axis_index("c")
        s = lax.axis_index("s")
        w = s * 2 + c
        pltpu.sync_copy(dstp.at[pl.ds(w * NB, NB)], dst_v)
        for k in range(K):
            pltpu.sync_copy(srcp.at[pl.ds((w * K + k) * EPW, EPW)], src_v)
            pltpu.sync_copy(zeros_hbm, acc.at[pl.ds(s * RPT, RPT)])
            plsc.subcore_barrier()

            def body(j, carry):
                idx = src_v.at[pl.ds(j * BT, BT)]
                pltpu.async_copy(table.at[idx], buf, sem).wait()
                pltpu.sync_copy(buf, acc.at[dst_v.at[j]], add=True)
                return carry

            lax.fori_loop(0, NB, body, 0)
            plsc.subcore_barrier()
            base = (c * K + k) * NROWS + s * RPT
            for z in range(RPT // BT):
                pltpu.sync_copy(acc.at[pl.ds(s * RPT + z * BT, BT)],
                                out.at[pl.ds(base + z * BT, BT)])
            plsc.subcore_barrier()

    return agg


_agg1 = _make_agg(1)
_agg16 = _make_agg(16)
_agg4 = _make_agg(4)

_deg_mesh = plsc.VectorSubcoreMesh(core_axis_name="c", subcore_axis_name="s")


@functools.partial(
    pl.kernel,
    out_type=jax.ShapeDtypeStruct((2 * NROWS, C), jnp.float32),
    mesh=_deg_mesh,
    scratch_types=[
        pltpu.VMEM((NB, BT), jnp.int32),
        pltpu.VMEM((BT, C), jnp.float32),
        pltpu.VMEM_SHARED((NROWS, C), jnp.float32),
        pltpu.SemaphoreType.DMA,
    ],
)
def _deg_kernel(dstp, ones_hbm, zeros_hbm, out, dst_v, obuf, acc, sem):
    del sem
    c = lax.axis_index("c")
    s = lax.axis_index("s")
    w = s * 2 + c
    pltpu.sync_copy(dstp.at[pl.ds(w * NB, NB)], dst_v)
    pltpu.sync_copy(ones_hbm, obuf)
    pltpu.sync_copy(zeros_hbm, acc.at[pl.ds(s * RPT, RPT)])
    plsc.subcore_barrier()

    def body(j, carry):
        pltpu.sync_copy(obuf, acc.at[dst_v.at[j]], add=True)
        return carry

    lax.fori_loop(0, NB, body, 0)
    plsc.subcore_barrier()
    for z in range(RPT // BT):
        pltpu.sync_copy(acc.at[pl.ds(s * RPT + z * BT, BT)],
                        out.at[pl.ds(c * NROWS + s * RPT + z * BT, BT)])


def _mm(a, b):
    return lax.dot_general(a, b, (((1,), (0,)), ((), ())),
                           preferred_element_type=jnp.float32)


def _ka_body(degp_ref, nodes_ref, W1_ref, dinv_ref, u1_ref):
    p = degp_ref[...]
    cnt = p[0, :, 0:1] + p[1, :, 0:1]
    dinv = lax.rsqrt(cnt + 1.0)
    dinv_ref[...] = dinv
    xw1 = _mm(nodes_ref[...], W1_ref[...])
    u1 = dinv * xw1
    for k in range(H // C):
        u1_ref[k] = u1[:, k * C:(k + 1) * C]


def _ka(degp, nodes, W1):
    TN = 1000
    return pl.pallas_call(
        _ka_body,
        grid=(N // TN,),
        in_specs=[
            pl.BlockSpec((2, TN, C), lambda j: (0, j, 0)),
            pl.BlockSpec((TN, D), lambda j: (j, 0)),
            pl.BlockSpec((D, H), lambda j: (0, 0)),
        ],
        out_specs=[
            pl.BlockSpec((TN, 1), lambda j: (j, 0)),
            pl.BlockSpec((H // C, TN, C), lambda j: (0, j, 0)),
        ],
        out_shape=[
            jax.ShapeDtypeStruct((N, 1), jnp.float32),
            jax.ShapeDtypeStruct((H // C, N, C), jnp.float32),
        ],
    )(degp, nodes, W1)


def _kb_body(zp_ref, u_ref, dinv_ref, W_ref, b_ref, uo_ref, *, kin, kout):
    dinv = dinv_ref[...]
    parts = [dinv * (zp_ref[0, k] + zp_ref[1, k] + u_ref[k])
             for k in range(kin)]
    h = jnp.maximum(jnp.concatenate(parts, axis=1) + b_ref[...], 0.0)
    xw = _mm(h, W_ref[...])
    uo = dinv * xw
    for k in range(kout):
        uo_ref[k] = uo[:, k * C:(k + 1) * C]


def _kbc(zp, u, dinv, W, b, kin, kout, TN):
    body = functools.partial(_kb_body, kin=kin, kout=kout)
    din = kin * C
    dout = kout * C
    return pl.pallas_call(
        body,
        grid=(N // TN,),
        in_specs=[
            pl.BlockSpec((2, kin, TN, C), lambda j: (0, 0, j, 0)),
            pl.BlockSpec((kin, TN, C), lambda j: (0, j, 0)),
            pl.BlockSpec((TN, 1), lambda j: (j, 0)),
            pl.BlockSpec((din, dout), lambda j: (0, 0)),
            pl.BlockSpec((1, din), lambda j: (0, 0)),
        ],
        out_specs=pl.BlockSpec((kout, TN, C), lambda j: (0, j, 0)),
        out_shape=jax.ShapeDtypeStruct((kout, N, C), jnp.float32),
    )(zp, u, dinv, W, b.reshape(1, din))


def _imgn_body(img_ref, out_ref):
    x = img_ref[...]
    nrm = jnp.sqrt(jnp.sum(x * x, axis=1, keepdims=True))
    out_ref[...] = x / jnp.maximum(nrm, 1e-12)


def _imgn(img):
    return pl.pallas_call(
        _imgn_body,
        grid=(1,),
        in_specs=[pl.BlockSpec((B, OUT), lambda j: (0, 0))],
        out_specs=pl.BlockSpec((B, OUT), lambda j: (0, 0)),
        out_shape=jax.ShapeDtypeStruct((B, OUT), jnp.float32),
    )(img)


def _k8_body(zp_ref, y3_ref, dinv_ref, b3_ref, imgn_ref, out_ref):
    dinv = dinv_ref[...]
    parts = [dinv * (zp_ref[0, k] + zp_ref[1, k] + y3_ref[k])
             for k in range(OUT // C)]
    out3 = jnp.concatenate(parts, axis=1) + b3_ref[...]
    nrm = jnp.sqrt(jnp.sum(out3 * out3, axis=1, keepdims=True))
    cur = out3 / jnp.maximum(nrm, 1e-12)
    out_ref[...] = lax.dot_general(imgn_ref[...], cur,
                                   (((1,), (1,)), ((), ())),
                                   preferred_element_type=jnp.float32)


def _k8(zp, y3, dinv, b3, imgn):
    TC_ = 1280
    return pl.pallas_call(
        _k8_body,
        grid=(NP // TC_,),
        in_specs=[
            pl.BlockSpec((2, OUT // C, TC_, C), lambda j: (0, 0, j, 0)),
            pl.BlockSpec((OUT // C, TC_, C), lambda j: (0, j, 0)),
            pl.BlockSpec((TC_, 1), lambda j: (j, 0)),
            pl.BlockSpec((1, OUT), lambda j: (0, 0)),
            pl.BlockSpec((B, OUT), lambda j: (0, 0)),
        ],
        out_specs=pl.BlockSpec((B, TC_), lambda j: (0, j)),
        out_shape=jax.ShapeDtypeStruct((B, NP), jnp.float32),
    )(zp, y3, dinv, b3.reshape(1, OUT), imgn)


def _chunked_src(src_pad, K):
    s = src_pad.reshape(NW, 1, EPW)
    return (s + (jnp.arange(K, dtype=jnp.int32) * N).reshape(1, K, 1)).reshape(-1)


def kernel(img, nodes, edge_index, W1, b1, W2, b2, W3, b3):
    src = edge_index[0]
    dst = edge_index[1]
    src_pad = jnp.concatenate(
        [src, jnp.zeros((EPAD - E,), jnp.int32)])
    dst_pad = jnp.concatenate(
        [dst, jnp.full((EPAD - E,), N, jnp.int32)])
    dstp = dst_pad.reshape(NW * NB, BT)

    zerosC = jnp.zeros((RPT, C), jnp.float32)
    onesC = jnp.zeros((BT, C), jnp.float32).at[:, 0].set(1.0)

    src16 = _chunked_src(src_pad, H // C)
    src4 = _chunked_src(src_pad, OUT // C)

    degp = _deg_kernel(dstp, onesC, zerosC).reshape(2, NROWS, C)
    dinv, u1 = _ka(degp, nodes, W1)

    z1 = _agg16(u1.reshape(H // C * N, C), src16, dstp, zerosC)
    z1 = z1.reshape(2, H // C, NROWS, C)
    u2 = _kbc(z1, u1, dinv, W2, b1, H // C, H // C, 200)

    z2 = _agg16(u2.reshape(H // C * N, C), src16, dstp, zerosC)
    z2 = z2.reshape(2, H // C, NROWS, C)
    u3 = _kbc(z2, u2, dinv, W3, b2, H // C, OUT // C, 200)

    z3 = _agg4(u3.reshape(OUT // C * N, C), src4, dstp, zerosC)
    z3 = z3.reshape(2, OUT // C, NROWS, C)

    imgn = _imgn(img)
    pair = _k8(z3, u3, dinv, b3, imgn)
    return pair[:, 500:N]


# speedup vs baseline: 1.7138x; 1.1223x over previous
import functools

import jax
import jax.numpy as jnp
from jax import lax
from jax.experimental import pallas as pl
from jax.experimental.pallas import tpu as pltpu
from jax.experimental.pallas import tpu_sc as plsc

N = 10000
D = 128
H = 2048
OUT = 512
B = 1024
E = 320000

C = 128
NW = 32
BT = 128
NB = 80
EPW = NB * BT
EPAD = NW * EPW
NROWS = 10240
RPT = NROWS // 16
NP = 10240


def _make_agg(K):
    mesh = plsc.VectorSubcoreMesh(core_axis_name="c", subcore_axis_name="s")

    @functools.partial(
        pl.kernel,
        out_type=jax.ShapeDtypeStruct((2 * K * NROWS, C), jnp.float32),
        mesh=mesh,
        scratch_types=[
            pltpu.VMEM((BT,), jnp.int32),
            pltpu.VMEM((BT,), jnp.int32),
            pltpu.VMEM((BT,), jnp.int32),
            pltpu.VMEM((BT,), jnp.int32),
            pltpu.VMEM((BT, C), jnp.float32),
            pltpu.VMEM((BT, C), jnp.float32),
            pltpu.VMEM_SHARED((NROWS, C), jnp.float32),
            pltpu.SemaphoreType.DMA,
            pltpu.SemaphoreType.DMA,
        ],
    )
    def agg(table, srcp, dstp, zeros_hbm, out,
            sbufA, dbufA, sbufB, dbufB, gbufA, gbufB, acc, semA, semB):
        c = lax.axis_index("c")
        s = lax.axis_index("s")
        w = s * 2 + c
        base_d = w * EPW
        for k in range(K):
            base_s = (w * K + k) * EPW
            pltpu.sync_copy(srcp.at[pl.ds(base_s, BT)], sbufA)
            pltpu.sync_copy(dstp.at[pl.ds(base_d, BT)], dbufA)
            pltpu.async_copy(table.at[sbufA], gbufA, semA)
            pltpu.sync_copy(srcp.at[pl.ds(base_s + BT, BT)], sbufB)
            pltpu.sync_copy(dstp.at[pl.ds(base_d + BT, BT)], dbufB)
            pltpu.async_copy(table.at[sbufB], gbufB, semB)
            pltpu.sync_copy(zeros_hbm, acc.at[pl.ds(s * RPT, RPT)])
            plsc.subcore_barrier()

            def body(p, carry):
                j0 = 2 * p
                pltpu.make_async_copy(table.at[sbufA], gbufA, semA).wait()
                pltpu.sync_copy(gbufA, acc.at[dbufA], add=True)

                @pl.when(j0 + 2 < NB)
                def _():
                    pltpu.sync_copy(
                        srcp.at[pl.ds(base_s + (j0 + 2) * BT, BT)], sbufA)
                    pltpu.sync_copy(
                        dstp.at[pl.ds(base_d + (j0 + 2) * BT, BT)], dbufA)
                    pltpu.async_copy(table.at[sbufA], gbufA, semA)

                pltpu.make_async_copy(table.at[sbufB], gbufB, semB).wait()
                pltpu.sync_copy(gbufB, acc.at[dbufB], add=True)

                @pl.when(j0 + 3 < NB)
                def _():
                    pltpu.sync_copy(
                        srcp.at[pl.ds(base_s + (j0 + 3) * BT, BT)], sbufB)
                    pltpu.sync_copy(
                        dstp.at[pl.ds(base_d + (j0 + 3) * BT, BT)], dbufB)
                    pltpu.async_copy(table.at[sbufB], gbufB, semB)

                return carry

            lax.fori_loop(0, NB // 2, body, 0)
            plsc.subcore_barrier()
            base = (c * K + k) * NROWS + s * RPT
            for z in range(RPT // BT):
                pltpu.sync_copy(acc.at[pl.ds(s * RPT + z * BT, BT)],
                                out.at[pl.ds(base + z * BT, BT)])
            plsc.subcore_barrier()

    return agg


_agg1 = _make_agg(1)
_agg16 = _make_agg(16)
_agg4 = _make_agg(4)

_deg_mesh = plsc.VectorSubcoreMesh(core_axis_name="c", subcore_axis_name="s")


@functools.partial(
    pl.kernel,
    out_type=jax.ShapeDtypeStruct((2 * NROWS, C), jnp.float32),
    mesh=_deg_mesh,
    scratch_types=[
        pltpu.VMEM((NB, BT), jnp.int32),
        pltpu.VMEM((BT, C), jnp.float32),
        pltpu.VMEM_SHARED((NROWS, C), jnp.float32),
        pltpu.SemaphoreType.DMA,
    ],
)
def _deg_kernel(dstp, ones_hbm, zeros_hbm, out, dst_v, obuf, acc, sem):
    del sem
    c = lax.axis_index("c")
    s = lax.axis_index("s")
    w = s * 2 + c
    pltpu.sync_copy(dstp.at[pl.ds(w * NB, NB)], dst_v)
    pltpu.sync_copy(ones_hbm, obuf)
    pltpu.sync_copy(zeros_hbm, acc.at[pl.ds(s * RPT, RPT)])
    plsc.subcore_barrier()

    def body(j, carry):
        pltpu.sync_copy(obuf, acc.at[dst_v.at[j]], add=True)
        return carry

    lax.fori_loop(0, NB, body, 0)
    plsc.subcore_barrier()
    for z in range(RPT // BT):
        pltpu.sync_copy(acc.at[pl.ds(s * RPT + z * BT, BT)],
                        out.at[pl.ds(c * NROWS + s * RPT + z * BT, BT)])


def _mm(a, b):
    return lax.dot_general(a, b, (((1,), (0,)), ((), ())),
                           preferred_element_type=jnp.float32)


def _ka_body(degp_ref, nodes_ref, W1_ref, dinv_ref, u1_ref):
    p = degp_ref[...]
    cnt = p[0, :, 0:1] + p[1, :, 0:1]
    dinv = lax.rsqrt(cnt + 1.0)
    dinv_ref[...] = dinv
    xw1 = _mm(nodes_ref[...], W1_ref[...])
    u1 = dinv * xw1
    for k in range(H // C):
        u1_ref[k] = u1[:, k * C:(k + 1) * C]


def _ka(degp, nodes, W1):
    TN = 1000
    return pl.pallas_call(
        _ka_body,
        grid=(N // TN,),
        in_specs=[
            pl.BlockSpec((2, TN, C), lambda j: (0, j, 0)),
            pl.BlockSpec((TN, D), lambda j: (j, 0)),
            pl.BlockSpec((D, H), lambda j: (0, 0)),
        ],
        out_specs=[
            pl.BlockSpec((TN, 1), lambda j: (j, 0)),
            pl.BlockSpec((H // C, TN, C), lambda j: (0, j, 0)),
        ],
        out_shape=[
            jax.ShapeDtypeStruct((N, 1), jnp.float32),
            jax.ShapeDtypeStruct((H // C, N, C), jnp.float32),
        ],
    )(degp, nodes, W1)


def _kb_body(zp_ref, u_ref, dinv_ref, W_ref, b_ref, uo_ref, *, kin, kout):
    dinv = dinv_ref[...]
    parts = [dinv * (zp_ref[0, k] + zp_ref[1, k] + u_ref[k])
             for k in range(kin)]
    h = jnp.maximum(jnp.concatenate(parts, axis=1) + b_ref[...], 0.0)
    xw = _mm(h, W_ref[...])
    uo = dinv * xw
    for k in range(kout):
        uo_ref[k] = uo[:, k * C:(k + 1) * C]


def _kbc(zp, u, dinv, W, b, kin, kout, TN):
    body = functools.partial(_kb_body, kin=kin, kout=kout)
    din = kin * C
    dout = kout * C
    return pl.pallas_call(
        body,
        grid=(N // TN,),
        in_specs=[
            pl.BlockSpec((2, kin, TN, C), lambda j: (0, 0, j, 0)),
            pl.BlockSpec((kin, TN, C), lambda j: (0, j, 0)),
            pl.BlockSpec((TN, 1), lambda j: (j, 0)),
            pl.BlockSpec((din, dout), lambda j: (0, 0)),
            pl.BlockSpec((1, din), lambda j: (0, 0)),
        ],
        out_specs=pl.BlockSpec((kout, TN, C), lambda j: (0, j, 0)),
        out_shape=jax.ShapeDtypeStruct((kout, N, C), jnp.float32),
    )(zp, u, dinv, W, b.reshape(1, din))


def _imgn_body(img_ref, out_ref):
    x = img_ref[...]
    nrm = jnp.sqrt(jnp.sum(x * x, axis=1, keepdims=True))
    out_ref[...] = x / jnp.maximum(nrm, 1e-12)


def _imgn(img):
    return pl.pallas_call(
        _imgn_body,
        grid=(1,),
        in_specs=[pl.BlockSpec((B, OUT), lambda j: (0, 0))],
        out_specs=pl.BlockSpec((B, OUT), lambda j: (0, 0)),
        out_shape=jax.ShapeDtypeStruct((B, OUT), jnp.float32),
    )(img)


def _k8_body(zp_ref, y3_ref, dinv_ref, b3_ref, imgn_ref, out_ref):
    dinv = dinv_ref[...]
    parts = [dinv * (zp_ref[0, k] + zp_ref[1, k] + y3_ref[k])
             for k in range(OUT // C)]
    out3 = jnp.concatenate(parts, axis=1) + b3_ref[...]
    nrm = jnp.sqrt(jnp.sum(out3 * out3, axis=1, keepdims=True))
    cur = out3 / jnp.maximum(nrm, 1e-12)
    out_ref[...] = lax.dot_general(imgn_ref[...], cur,
                                   (((1,), (1,)), ((), ())),
                                   preferred_element_type=jnp.float32)


def _k8(zp, y3, dinv, b3, imgn):
    TC_ = 1280
    return pl.pallas_call(
        _k8_body,
        grid=(NP // TC_,),
        in_specs=[
            pl.BlockSpec((2, OUT // C, TC_, C), lambda j: (0, 0, j, 0)),
            pl.BlockSpec((OUT // C, TC_, C), lambda j: (0, j, 0)),
            pl.BlockSpec((TC_, 1), lambda j: (j, 0)),
            pl.BlockSpec((1, OUT), lambda j: (0, 0)),
            pl.BlockSpec((B, OUT), lambda j: (0, 0)),
        ],
        out_specs=pl.BlockSpec((B, TC_), lambda j: (0, j)),
        out_shape=jax.ShapeDtypeStruct((B, NP), jnp.float32),
    )(zp, y3, dinv, b3.reshape(1, OUT), imgn)


def _chunked_src(src_pad, K):
    s = src_pad.reshape(NW, 1, EPW)
    return (s + (jnp.arange(K, dtype=jnp.int32) * N).reshape(1, K, 1)).reshape(-1)


def kernel(img, nodes, edge_index, W1, b1, W2, b2, W3, b3):
    src = edge_index[0]
    dst = edge_index[1]
    src_pad = jnp.concatenate(
        [src, jnp.zeros((EPAD - E,), jnp.int32)])
    dst_pad = jnp.concatenate(
        [dst, jnp.full((EPAD - E,), N, jnp.int32)])
    dstp = dst_pad.reshape(NW * NB, BT)

    zerosC = jnp.zeros((RPT, C), jnp.float32)
    onesC = jnp.zeros((BT, C), jnp.float32).at[:, 0].set(1.0)

    src16 = _chunked_src(src_pad, H // C)
    src4 = _chunked_src(src_pad, OUT // C)

    degp = _deg_kernel(dstp, onesC, zerosC).reshape(2, NROWS, C)
    dinv, u1 = _ka(degp, nodes, W1)

    z1 = _agg16(u1.reshape(H // C * N, C), src16, dst_pad, zerosC)
    z1 = z1.reshape(2, H // C, NROWS, C)
    u2 = _kbc(z1, u1, dinv, W2, b1, H // C, H // C, 200)

    z2 = _agg16(u2.reshape(H // C * N, C), src16, dst_pad, zerosC)
    z2 = z2.reshape(2, H // C, NROWS, C)
    u3 = _kbc(z2, u2, dinv, W3, b2, H // C, OUT // C, 200)

    z3 = _agg4(u3.reshape(OUT // C * N, C), src4, dst_pad, zerosC)
    z3 = z3.reshape(2, OUT // C, NROWS, C)

    imgn = _imgn(img)
    pair = _k8(z3, u3, dinv, b3, imgn)
    return pair[:, 500:N]
